# baseline (speedup 1.0000x reference)
import functools

import jax
import jax.numpy as jnp
from jax import lax
from jax.experimental import pallas as pl
from jax.experimental.pallas import tpu as pltpu
from jax.experimental.pallas import tpu_sc as plsc

HEIGHT = 1024
WIDTH = 1024
B = 8
N = 262144
HW = HEIGHT * WIDTH
NC = 2
NS = 16
CHUNK = 2048
PER_TILE = N // NS
SLICE = HW // NS
RANK0 = 2621
FRAC = 0.43


def _params_body(x_ref, y_ref, z_ref, o_ref):
    u = jnp.uint32
    inf = jnp.float32(jnp.inf)

    def minmax(ref):
        def mm(k, carry):
            lo, hi = carry
            blk = ref[0, pl.ds(k * 64, 64), :]
            return jnp.minimum(lo, blk), jnp.maximum(hi, blk)

        lo, hi = lax.fori_loop(
            0, 32, mm, (jnp.full((64, 128), inf, jnp.float32),
                        jnp.full((64, 128), -inf, jnp.float32)), unroll=4)
        return jnp.min(lo), jnp.max(hi)

    xmin, xmax = minmax(x_ref)
    ymin, ymax = minmax(y_ref)
    zmin, zmax = minmax(z_ref)
    xspan = xmax - xmin
    yspan = ymax - ymin
    zspan = zmax - zmin

    def tofl(k):
        return lax.bitcast_convert_type(
            jnp.where(k >= u(0x80000000), k ^ u(0x80000000), ~k), jnp.float32)

    def bis(_, carry):
        lo1, hi1 = carry
        mid1 = lo1 + (hi1 - lo1) // u(2)
        midf = tofl(mid1)

        def inner(k, a1):
            blk = z_ref[0, pl.ds(k * 64, 64), :]
            return a1 + (blk <= midf).astype(jnp.int32)

        a1 = lax.fori_loop(0, 32, inner, jnp.zeros((64, 128), jnp.int32),
                           unroll=4)
        c1 = jnp.sum(a1)
        t1 = c1 >= RANK0 + 1
        return (jnp.where(t1, lo1, mid1 + u(1)), jnp.where(t1, mid1, hi1))

    k1, _ = lax.fori_loop(0, 32, bis, (u(0), u(0xFFFFFFFF)))
    z1 = tofl(k1)

    def nxt(k, carry):
        accm, accc = carry
        blk = z_ref[0, pl.ds(k * 64, 64), :]
        gt = blk > z1
        return (jnp.minimum(accm, jnp.where(gt, blk, inf)),
                accc + (~gt).astype(jnp.int32))

    accm, accc = lax.fori_loop(
        0, 32, nxt, (jnp.full((64, 128), inf, jnp.float32),
                     jnp.zeros((64, 128), jnp.int32)), unroll=4)
    z2 = jnp.min(accm)
    z2 = jnp.where(jnp.sum(accc) >= RANK0 + 2, z1,
                   jnp.where(jnp.isfinite(z2), z2, z1))

    d1 = (z1 - zmin) / zspan
    d2 = (z2 - zmin) / zspan
    q = d1 + jnp.float32(FRAC) * (d2 - d1)

    rsc = jnp.float32(1022.0) / xspan
    csc = jnp.float32(1022.0) / yspan
    zsc = jnp.float32(1.0) / zspan
    rows = lax.broadcasted_iota(jnp.int32, (8, 128), 0)
    acc = jnp.zeros((8, 128), jnp.float32)
    for k, p in enumerate([xmin, rsc, ymin, csc, zmin, zsc, q,
                           jnp.float32(0.0)]):
        acc = jnp.where(rows == k, p, acc)
    o_ref[0] = acc


_params_tc = pl.pallas_call(
    _params_body,
    grid=(B,),
    in_specs=[pl.BlockSpec((1, 2048, 128), lambda b: (b, 0, 0))] * 3,
    out_specs=pl.BlockSpec((1, 8, 128), lambda b: (b, 0, 0)),
    out_shape=jax.ShapeDtypeStruct((B, 8, 128), jnp.float32),
)


NSUB = PER_TILE // CHUNK
NROW = 4 * CHUNK // 128


def _splat_body(xs, ys, zs, pr, out, raster, xv, yv, zv, pv, idxb, valb, zb,
                sem_in, sem_sc):
    c = lax.axis_index("c")
    sid = lax.axis_index("s")
    zero16 = jnp.zeros((16,), jnp.float32)

    @pl.loop(0, zb.shape[0] // 16)
    def _zero(i):
        zb[pl.ds(i * 16, 16)] = zero16

    def fire_loads(gb, row0, p):
        src = pl.ds(row0, CHUNK // 128)
        pltpu.async_copy(xs.at[gb, src, :], xv.at[p], sem_in)
        pltpu.async_copy(ys.at[gb, src, :], yv.at[p], sem_in)
        pltpu.async_copy(zs.at[gb, src, :], zv.at[p], sem_in)

    def drain_loads(gb, row0, p):
        src = pl.ds(row0, CHUNK // 128)
        pltpu.make_async_copy(xs.at[gb, src, :], xv.at[p], sem_in).wait()
        pltpu.make_async_copy(ys.at[gb, src, :], yv.at[p], sem_in).wait()
        pltpu.make_async_copy(zs.at[gb, src, :], zv.at[p], sem_in).wait()

    def fire_scatters(p):
        @pl.loop(0, NROW, unroll=4)
        def _f(jj):
            pltpu.async_copy(valb.at[p * NROW + jj],
                             raster.at[idxb.at[p * NROW + jj]],
                             sem_sc, add=True)

    def drain_scatters(p):
        @pl.loop(0, NROW, unroll=4)
        def _d(jj):
            pltpu.make_async_copy(valb.at[p * NROW + jj],
                                  raster.at[idxb.at[p * NROW + jj]],
                                  sem_sc).wait()

    @pl.loop(0, B // NC)
    def _batch(b):
        gb = c * (B // NC) + b
        trow = sid * (PER_TILE // 128)

        @pl.loop(0, SLICE // 8192)
        def _z8(z8):
            pltpu.sync_copy(zb, raster.at[pl.ds(sid * SLICE + z8 * 8192,
                                                8192)])

        plsc.subcore_barrier()

        pltpu.sync_copy(pr.at[pl.ds(gb * 128, 128)], pv)
        xmin = pv[pl.ds(0, 16)]
        rsc = pv[pl.ds(16, 16)]
        ymin = pv[pl.ds(32, 16)]
        csc = pv[pl.ds(48, 16)]
        zmin = pv[pl.ds(64, 16)]
        zsc = pv[pl.ds(80, 16)]
        q = pv[pl.ds(96, 16)]

        fire_loads(gb, trow, 0)

        @pl.loop(0, NSUB)
        def _sub(s8):
            p = s8 & 1
            drain_loads(gb, trow + s8 * (CHUNK // 128), p)

            @pl.when(s8 + 1 < NSUB)
            def _pref():
                fire_loads(gb, trow + (s8 + 1) * (CHUNK // 128), 1 - p)

            @pl.when(s8 >= 2)
            def _dr():
                drain_scatters(p)

            @pl.loop(0, CHUNK // 128)
            def _fill(j):
                for t in range(8):
                    cds = pl.ds(t * 16, 16)
                    x = xv[p, j, cds]
                    y = yv[p, j, cds]
                    z = zv[p, j, cds]
                    r = jnp.minimum((x - xmin) * rsc + 1.0, 1023.0)
                    cc = jnp.minimum((y - ymin) * csc + 1.0, 1023.0)
                    dist = (z - zmin) * zsc
                    s = 1.0 - jnp.maximum(dist, q)
                    flr = r.astype(jnp.int32)
                    flrf = flr.astype(jnp.float32)
                    cer = flr + jnp.where(flrf == r, 0, 1)
                    cerf = cer.astype(jnp.float32)
                    flc = cc.astype(jnp.int32)
                    flcf = flc.astype(jnp.float32)
                    cec = flc + jnp.where(flcf == cc, 0, 1)
                    cecf = cec.astype(jnp.float32)
                    sw0 = s * (r - flrf)
                    sw1 = s * (cerf - r)
                    wc0 = cc - flcf
                    wc1 = cecf - cc
                    r0 = flr * 1024
                    r1 = cer * 1024
                    ds = pl.ds(t * 16, 16)
                    G = CHUNK // 128
                    row = p * NROW + j
                    idxb[row, ds] = r0 + flc
                    idxb[G + row, ds] = r1 + flc
                    idxb[2 * G + row, ds] = r0 + cec
                    idxb[3 * G + row, ds] = r1 + cec
                    valb[row, ds] = sw0 * wc0
                    valb[G + row, ds] = sw1 * wc0
                    valb[2 * G + row, ds] = sw0 * wc1
                    valb[3 * G + row, ds] = sw1 * wc1

            fire_scatters(p)

        drain_scatters(0)
        drain_scatters(1)
        plsc.subcore_barrier()
        for k in range(SLICE // WIDTH):
            pltpu.async_copy(
                raster.at[pl.ds(sid * SLICE + k * WIDTH, WIDTH)],
                out.at[gb, pl.ds(sid * SLICE + k * WIDTH, WIDTH)], sem_in)
        for k in range(SLICE // WIDTH):
            pltpu.make_async_copy(
                raster.at[pl.ds(sid * SLICE + k * WIDTH, WIDTH)],
                out.at[gb, pl.ds(sid * SLICE + k * WIDTH, WIDTH)],
                sem_in).wait()


@functools.cache
def _splat_sc():
    return functools.partial(
        pl.kernel,
        out_type=jax.ShapeDtypeStruct((B, HW), jnp.float32),
        compiler_params=pltpu.CompilerParams(use_tc_tiling_on_sc=True),
        mesh=plsc.VectorSubcoreMesh(core_axis_name="c", subcore_axis_name="s",
                                    num_cores=NC, num_subcores=NS),
        scratch_types=[
            pltpu.VMEM_SHARED((HW,), jnp.float32),
            pltpu.VMEM((2, CHUNK // 128, 128), jnp.float32),
            pltpu.VMEM((2, CHUNK // 128, 128), jnp.float32),
            pltpu.VMEM((2, CHUNK // 128, 128), jnp.float32),
            pltpu.VMEM((128,), jnp.float32),
            pltpu.VMEM((2 * NROW, 128), jnp.int32),
            pltpu.VMEM((2 * NROW, 128), jnp.float32),
            pltpu.VMEM((8192,), jnp.float32),
            pltpu.SemaphoreType.DMA,
            pltpu.SemaphoreType.DMA,
        ],
    )(_splat_body)


def kernel(point_clouds):
    pct = jnp.swapaxes(point_clouds, 1, 2)
    xs = pct[:, 0, :].reshape(B, 2048, 128)
    ys = pct[:, 1, :].reshape(B, 2048, 128)
    zs = pct[:, 2, :].reshape(B, 2048, 128)
    params = _params_tc(xs, ys, zs)
    params_sc = params[:, :, :16].reshape(-1)
    flat = _splat_sc()(xs, ys, zs, params_sc)
    return flat.reshape(B, HEIGHT, WIDTH)


# speedup vs baseline: 1.0954x; 1.0954x over previous
import functools

import jax
import jax.numpy as jnp
from jax import lax
from jax.experimental import pallas as pl
from jax.experimental.pallas import tpu as pltpu
from jax.experimental.pallas import tpu_sc as plsc

HEIGHT = 1024
WIDTH = 1024
B = 8
N = 262144
HW = HEIGHT * WIDTH
NC = 2
NS = 16
CHUNK = 2048
PER_TILE = N // NS
SLICE = HW // NS
RANK0 = 2621
FRAC = 0.43


def _params_body(x_ref, y_ref, z_ref, o_ref):
    u = jnp.uint32
    inf = jnp.float32(jnp.inf)

    def minmax(ref):
        def mm(k, carry):
            lo, hi = carry
            blk = ref[0, pl.ds(k * 64, 64), :]
            return jnp.minimum(lo, blk), jnp.maximum(hi, blk)

        lo, hi = lax.fori_loop(
            0, 32, mm, (jnp.full((64, 128), inf, jnp.float32),
                        jnp.full((64, 128), -inf, jnp.float32)), unroll=4)
        return jnp.min(lo), jnp.max(hi)

    xmin, xmax = minmax(x_ref)
    ymin, ymax = minmax(y_ref)
    zmin, zmax = minmax(z_ref)
    xspan = xmax - xmin
    yspan = ymax - ymin
    zspan = zmax - zmin

    def tofl(k):
        return lax.bitcast_convert_type(
            jnp.where(k >= u(0x80000000), k ^ u(0x80000000), ~k), jnp.float32)

    def bis(_, carry):
        lo1, hi1 = carry
        mid1 = lo1 + (hi1 - lo1) // u(2)
        midf = tofl(mid1)

        def inner(k, a1):
            blk = z_ref[0, pl.ds(k * 64, 64), :]
            return a1 + (blk <= midf).astype(jnp.int32)

        a1 = lax.fori_loop(0, 32, inner, jnp.zeros((64, 128), jnp.int32),
                           unroll=4)
        c1 = jnp.sum(a1)
        t1 = c1 >= RANK0 + 1
        return (jnp.where(t1, lo1, mid1 + u(1)), jnp.where(t1, mid1, hi1))

    k1, _ = lax.fori_loop(0, 32, bis, (u(0), u(0xFFFFFFFF)))
    z1 = tofl(k1)

    def nxt(k, carry):
        accm, accc = carry
        blk = z_ref[0, pl.ds(k * 64, 64), :]
        gt = blk > z1
        return (jnp.minimum(accm, jnp.where(gt, blk, inf)),
                accc + (~gt).astype(jnp.int32))

    accm, accc = lax.fori_loop(
        0, 32, nxt, (jnp.full((64, 128), inf, jnp.float32),
                     jnp.zeros((64, 128), jnp.int32)), unroll=4)
    z2 = jnp.min(accm)
    z2 = jnp.where(jnp.sum(accc) >= RANK0 + 2, z1,
                   jnp.where(jnp.isfinite(z2), z2, z1))

    d1 = (z1 - zmin) / zspan
    d2 = (z2 - zmin) / zspan
    q = d1 + jnp.float32(FRAC) * (d2 - d1)

    rsc = jnp.float32(1022.0) / xspan
    csc = jnp.float32(1022.0) / yspan
    zsc = jnp.float32(1.0) / zspan
    rows = lax.broadcasted_iota(jnp.int32, (8, 128), 0)
    acc = jnp.zeros((8, 128), jnp.float32)
    for k, p in enumerate([xmin, rsc, ymin, csc, zmin, zsc, q,
                           jnp.float32(0.0)]):
        acc = jnp.where(rows == k, p, acc)
    o_ref[0] = acc


_params_tc = pl.pallas_call(
    _params_body,
    grid=(B,),
    in_specs=[pl.BlockSpec((1, 2048, 128), lambda b: (b, 0, 0))] * 3,
    out_specs=pl.BlockSpec((1, 8, 128), lambda b: (b, 0, 0)),
    out_shape=jax.ShapeDtypeStruct((B, 8, 128), jnp.float32),
)


NSUB = PER_TILE // CHUNK
NROW = 4 * CHUNK // 128


def _splat_body(xs, ys, zs, pr, out, raster, xv, yv, zv, pv, idxb, valb, zb,
                sem_in, sem_sc):
    c = lax.axis_index("c")
    sid = lax.axis_index("s")
    zero16 = jnp.zeros((16,), jnp.float32)

    @pl.loop(0, zb.shape[0] // 16)
    def _zero(i):
        zb[pl.ds(i * 16, 16)] = zero16

    def fire_loads(gb, row0, p):
        src = pl.ds(row0, CHUNK // 128)
        pltpu.async_copy(xs.at[gb, src, :], xv.at[p], sem_in)
        pltpu.async_copy(ys.at[gb, src, :], yv.at[p], sem_in)
        pltpu.async_copy(zs.at[gb, src, :], zv.at[p], sem_in)

    def drain_loads(gb, row0, p):
        src = pl.ds(row0, CHUNK // 128)
        pltpu.make_async_copy(xs.at[gb, src, :], xv.at[p], sem_in).wait()
        pltpu.make_async_copy(ys.at[gb, src, :], yv.at[p], sem_in).wait()
        pltpu.make_async_copy(zs.at[gb, src, :], zv.at[p], sem_in).wait()

    def fire_scatters(p):
        @pl.loop(0, NROW, unroll=4)
        def _f(jj):
            pltpu.async_copy(valb.at[p * NROW + jj],
                             raster.at[idxb.at[p * NROW + jj]],
                             sem_sc, add=True)

    def drain_scatters(p):
        @pl.loop(0, NROW, unroll=4)
        def _d(jj):
            pltpu.make_async_copy(valb.at[p * NROW + jj],
                                  raster.at[idxb.at[p * NROW + jj]],
                                  sem_sc).wait()

    @pl.loop(0, B // NC)
    def _batch(b):
        gb = c * (B // NC) + b
        trow = sid * (PER_TILE // 128)

        @pl.loop(0, SLICE // 8192)
        def _z8(z8):
            pltpu.sync_copy(zb, raster.at[pl.ds(sid * SLICE + z8 * 8192,
                                                8192)])

        plsc.subcore_barrier()

        pltpu.sync_copy(pr.at[pl.ds(gb * 128, 128)], pv)
        xmin = pv[pl.ds(0, 16)]
        rsc = pv[pl.ds(16, 16)]
        ymin = pv[pl.ds(32, 16)]
        csc = pv[pl.ds(48, 16)]
        zmin = pv[pl.ds(64, 16)]
        zsc = pv[pl.ds(80, 16)]
        q = pv[pl.ds(96, 16)]

        fire_loads(gb, trow, 0)

        @pl.loop(0, NSUB)
        def _sub(s8):
            p = s8 & 1
            drain_loads(gb, trow + s8 * (CHUNK // 128), p)

            @pl.when(s8 + 1 < NSUB)
            def _pref():
                fire_loads(gb, trow + (s8 + 1) * (CHUNK // 128), 1 - p)

            @pl.when(s8 >= 2)
            def _dr():
                drain_scatters(p)

            @pl.loop(0, CHUNK // 128)
            def _fill(j):
                for t in range(8):
                    cds = pl.ds(t * 16, 16)
                    x = xv[p, j, cds]
                    y = yv[p, j, cds]
                    z = zv[p, j, cds]
                    r = jnp.minimum((x - xmin) * rsc + 1.0, 1023.0)
                    cc = jnp.minimum((y - ymin) * csc + 1.0, 1023.0)
                    dist = (z - zmin) * zsc
                    s = 1.0 - jnp.maximum(dist, q)
                    flr = r.astype(jnp.int32)
                    flrf = flr.astype(jnp.float32)
                    cer = flr + jnp.where(flrf == r, 0, 1)
                    cerf = cer.astype(jnp.float32)
                    flc = cc.astype(jnp.int32)
                    flcf = flc.astype(jnp.float32)
                    cec = flc + jnp.where(flcf == cc, 0, 1)
                    cecf = cec.astype(jnp.float32)
                    sw0 = s * (r - flrf)
                    sw1 = s * (cerf - r)
                    wc0 = cc - flcf
                    wc1 = cecf - cc
                    r0 = flr * 1024
                    r1 = cer * 1024
                    ds = pl.ds(t * 16, 16)
                    G = CHUNK // 128
                    row = p * NROW + j
                    idxb[row, ds] = r0 + flc
                    idxb[G + row, ds] = r1 + flc
                    idxb[2 * G + row, ds] = r0 + cec
                    idxb[3 * G + row, ds] = r1 + cec
                    valb[row, ds] = sw0 * wc0
                    valb[G + row, ds] = sw1 * wc0
                    valb[2 * G + row, ds] = sw0 * wc1
                    valb[3 * G + row, ds] = sw1 * wc1

            fire_scatters(p)

        drain_scatters(0)
        drain_scatters(1)
        plsc.subcore_barrier()
        for k in range(SLICE // WIDTH):
            pltpu.async_copy(raster.at[pl.ds(sid * SLICE + k * WIDTH, WIDTH)],
                             out.at[gb, sid * (SLICE // WIDTH) + k, :],
                             sem_in)
        for k in range(SLICE // WIDTH):
            pltpu.make_async_copy(
                raster.at[pl.ds(sid * SLICE + k * WIDTH, WIDTH)],
                out.at[gb, sid * (SLICE // WIDTH) + k, :], sem_in).wait()


@functools.cache
def _splat_sc():
    return functools.partial(
        pl.kernel,
        out_type=jax.ShapeDtypeStruct((B, HEIGHT, WIDTH), jnp.float32),
        compiler_params=pltpu.CompilerParams(use_tc_tiling_on_sc=True),
        mesh=plsc.VectorSubcoreMesh(core_axis_name="c", subcore_axis_name="s",
                                    num_cores=NC, num_subcores=NS),
        scratch_types=[
            pltpu.VMEM_SHARED((HW,), jnp.float32),
            pltpu.VMEM((2, CHUNK // 128, 128), jnp.float32),
            pltpu.VMEM((2, CHUNK // 128, 128), jnp.float32),
            pltpu.VMEM((2, CHUNK // 128, 128), jnp.float32),
            pltpu.VMEM((128,), jnp.float32),
            pltpu.VMEM((2 * NROW, 128), jnp.int32),
            pltpu.VMEM((2 * NROW, 128), jnp.float32),
            pltpu.VMEM((8192,), jnp.float32),
            pltpu.SemaphoreType.DMA,
            pltpu.SemaphoreType.DMA,
        ],
    )(_splat_body)


def kernel(point_clouds):
    pct = jnp.swapaxes(point_clouds, 1, 2)
    xs = pct[:, 0, :].reshape(B, 2048, 128)
    ys = pct[:, 1, :].reshape(B, 2048, 128)
    zs = pct[:, 2, :].reshape(B, 2048, 128)
    params = _params_tc(xs, ys, zs)
    params_sc = params[:, :, :16].reshape(-1)
    return _splat_sc()(xs, ys, zs, params_sc)


# speedup vs baseline: 1.1095x; 1.0129x over previous
import functools

import jax
import jax.numpy as jnp
from jax import lax
from jax.experimental import pallas as pl
from jax.experimental.pallas import tpu as pltpu
from jax.experimental.pallas import tpu_sc as plsc

HEIGHT = 1024
WIDTH = 1024
B = 8
N = 262144
HW = HEIGHT * WIDTH
NC = 2
NS = 16
CHUNK = 2048
PER_TILE = N // NS
SLICE = HW // NS
RANK0 = 2621
FRAC = 0.43


def _params_body(x_ref, y_ref, z_ref, o_ref):
    u = jnp.uint32
    inf = jnp.float32(jnp.inf)

    def minmax(ref):
        def mm(k, carry):
            lo, hi = carry
            blk = ref[0, pl.ds(k * 64, 64), :]
            return jnp.minimum(lo, blk), jnp.maximum(hi, blk)

        lo, hi = lax.fori_loop(
            0, 32, mm, (jnp.full((64, 128), inf, jnp.float32),
                        jnp.full((64, 128), -inf, jnp.float32)), unroll=4)
        return jnp.min(lo), jnp.max(hi)

    xmin, xmax = minmax(x_ref)
    ymin, ymax = minmax(y_ref)
    zmin, zmax = minmax(z_ref)
    xspan = xmax - xmin
    yspan = ymax - ymin
    zspan = zmax - zmin

    def tofl(k):
        return lax.bitcast_convert_type(
            jnp.where(k >= u(0x80000000), k ^ u(0x80000000), ~k), jnp.float32)

    def bis(_, carry):
        lo1, hi1 = carry
        mid1 = lo1 + (hi1 - lo1) // u(2)
        midf = tofl(mid1)

        def inner(k, a1):
            blk = z_ref[0, pl.ds(k * 64, 64), :]
            return a1 + (blk <= midf).astype(jnp.int32)

        a1 = lax.fori_loop(0, 32, inner, jnp.zeros((64, 128), jnp.int32),
                           unroll=8)
        c1 = jnp.sum(a1)
        t1 = c1 >= RANK0 + 1
        return (jnp.where(t1, lo1, mid1 + u(1)), jnp.where(t1, mid1, hi1))

    k1, _ = lax.fori_loop(0, 32, bis, (u(0), u(0xFFFFFFFF)))
    z1 = tofl(k1)

    def nxt(k, carry):
        accm, accc = carry
        blk = z_ref[0, pl.ds(k * 64, 64), :]
        gt = blk > z1
        return (jnp.minimum(accm, jnp.where(gt, blk, inf)),
                accc + (~gt).astype(jnp.int32))

    accm, accc = lax.fori_loop(
        0, 32, nxt, (jnp.full((64, 128), inf, jnp.float32),
                     jnp.zeros((64, 128), jnp.int32)), unroll=4)
    z2 = jnp.min(accm)
    z2 = jnp.where(jnp.sum(accc) >= RANK0 + 2, z1,
                   jnp.where(jnp.isfinite(z2), z2, z1))

    d1 = (z1 - zmin) / zspan
    d2 = (z2 - zmin) / zspan
    q = d1 + jnp.float32(FRAC) * (d2 - d1)

    rsc = jnp.float32(1022.0) / xspan
    csc = jnp.float32(1022.0) / yspan
    zsc = jnp.float32(1.0) / zspan
    rows = lax.broadcasted_iota(jnp.int32, (8, 128), 0)
    acc = jnp.zeros((8, 128), jnp.float32)
    for k, p in enumerate([xmin, rsc, ymin, csc, zmin, zsc, q,
                           jnp.float32(0.0)]):
        acc = jnp.where(rows == k, p, acc)
    o_ref[0] = acc


_params_tc = pl.pallas_call(
    _params_body,
    grid=(B,),
    in_specs=[pl.BlockSpec((1, 2048, 128), lambda b: (b, 0, 0))] * 3,
    out_specs=pl.BlockSpec((1, 8, 128), lambda b: (b, 0, 0)),
    out_shape=jax.ShapeDtypeStruct((B, 8, 128), jnp.float32),
)


NSUB = PER_TILE // CHUNK
NROW = 4 * CHUNK // 128


def _splat_body(xs, ys, zs, pr, out, raster, xv, yv, zv, pv, idxb, valb, zb,
                sem_in, sem_sc):
    c = lax.axis_index("c")
    sid = lax.axis_index("s")
    zero16 = jnp.zeros((16,), jnp.float32)

    @pl.loop(0, zb.shape[0] // 16)
    def _zero(i):
        zb[pl.ds(i * 16, 16)] = zero16

    def fire_loads(gb, row0, p):
        src = pl.ds(row0, CHUNK // 128)
        pltpu.async_copy(xs.at[gb, src, :], xv.at[p], sem_in)
        pltpu.async_copy(ys.at[gb, src, :], yv.at[p], sem_in)
        pltpu.async_copy(zs.at[gb, src, :], zv.at[p], sem_in)

    def drain_loads(gb, row0, p):
        src = pl.ds(row0, CHUNK // 128)
        pltpu.make_async_copy(xs.at[gb, src, :], xv.at[p], sem_in).wait()
        pltpu.make_async_copy(ys.at[gb, src, :], yv.at[p], sem_in).wait()
        pltpu.make_async_copy(zs.at[gb, src, :], zv.at[p], sem_in).wait()

    def fire_scatters(p):
        @pl.loop(0, NROW, unroll=4)
        def _f(jj):
            pltpu.async_copy(valb.at[p * NROW + jj],
                             raster.at[idxb.at[p * NROW + jj]],
                             sem_sc, add=True)

    def drain_scatters(p):
        @pl.loop(0, NROW, unroll=4)
        def _d(jj):
            pltpu.make_async_copy(valb.at[p * NROW + jj],
                                  raster.at[idxb.at[p * NROW + jj]],
                                  sem_sc).wait()

    @pl.loop(0, B // NC)
    def _batch(b):
        gb = c * (B // NC) + b
        trow = sid * (PER_TILE // 128)

        @pl.loop(0, SLICE // 8192)
        def _z8(z8):
            pltpu.sync_copy(zb, raster.at[pl.ds(sid * SLICE + z8 * 8192,
                                                8192)])

        plsc.subcore_barrier()

        pltpu.sync_copy(pr.at[pl.ds(gb * 128, 128)], pv)
        xmin = pv[pl.ds(0, 16)]
        rsc = pv[pl.ds(16, 16)]
        ymin = pv[pl.ds(32, 16)]
        csc = pv[pl.ds(48, 16)]
        zmin = pv[pl.ds(64, 16)]
        zsc = pv[pl.ds(80, 16)]
        q = pv[pl.ds(96, 16)]

        fire_loads(gb, trow, 0)

        @pl.loop(0, NSUB)
        def _sub(s8):
            p = s8 & 1
            drain_loads(gb, trow + s8 * (CHUNK // 128), p)

            @pl.when(s8 + 1 < NSUB)
            def _pref():
                fire_loads(gb, trow + (s8 + 1) * (CHUNK // 128), 1 - p)

            @pl.when(s8 >= 2)
            def _dr():
                drain_scatters(p)

            @pl.loop(0, CHUNK // 128)
            def _fill(j):
                for t in range(8):
                    cds = pl.ds(t * 16, 16)
                    x = xv[p, j, cds]
                    y = yv[p, j, cds]
                    z = zv[p, j, cds]
                    r = jnp.minimum((x - xmin) * rsc + 1.0, 1023.0)
                    cc = jnp.minimum((y - ymin) * csc + 1.0, 1023.0)
                    dist = (z - zmin) * zsc
                    s = 1.0 - jnp.maximum(dist, q)
                    flr = r.astype(jnp.int32)
                    flrf = flr.astype(jnp.float32)
                    cer = flr + jnp.where(flrf == r, 0, 1)
                    cerf = cer.astype(jnp.float32)
                    flc = cc.astype(jnp.int32)
                    flcf = flc.astype(jnp.float32)
                    cec = flc + jnp.where(flcf == cc, 0, 1)
                    cecf = cec.astype(jnp.float32)
                    sw0 = s * (r - flrf)
                    sw1 = s * (cerf - r)
                    wc0 = cc - flcf
                    wc1 = cecf - cc
                    r0 = flr * 1024
                    r1 = cer * 1024
                    ds = pl.ds(t * 16, 16)
                    G = CHUNK // 128
                    row = p * NROW + j
                    idxb[row, ds] = r0 + flc
                    idxb[G + row, ds] = r1 + flc
                    idxb[2 * G + row, ds] = r0 + cec
                    idxb[3 * G + row, ds] = r1 + cec
                    valb[row, ds] = sw0 * wc0
                    valb[G + row, ds] = sw1 * wc0
                    valb[2 * G + row, ds] = sw0 * wc1
                    valb[3 * G + row, ds] = sw1 * wc1

            fire_scatters(p)

        drain_scatters(0)
        drain_scatters(1)
        plsc.subcore_barrier()
        for k in range(SLICE // WIDTH):
            pltpu.async_copy(raster.at[pl.ds(sid * SLICE + k * WIDTH, WIDTH)],
                             out.at[gb, sid * (SLICE // WIDTH) + k, :],
                             sem_in)
        for k in range(SLICE // WIDTH):
            pltpu.make_async_copy(
                raster.at[pl.ds(sid * SLICE + k * WIDTH, WIDTH)],
                out.at[gb, sid * (SLICE // WIDTH) + k, :], sem_in).wait()


@functools.cache
def _splat_sc():
    return functools.partial(
        pl.kernel,
        out_type=jax.ShapeDtypeStruct((B, HEIGHT, WIDTH), jnp.float32),
        compiler_params=pltpu.CompilerParams(use_tc_tiling_on_sc=True),
        mesh=plsc.VectorSubcoreMesh(core_axis_name="c", subcore_axis_name="s",
                                    num_cores=NC, num_subcores=NS),
        scratch_types=[
            pltpu.VMEM_SHARED((HW,), jnp.float32),
            pltpu.VMEM((2, CHUNK // 128, 128), jnp.float32),
            pltpu.VMEM((2, CHUNK // 128, 128), jnp.float32),
            pltpu.VMEM((2, CHUNK // 128, 128), jnp.float32),
            pltpu.VMEM((128,), jnp.float32),
            pltpu.VMEM((2 * NROW, 128), jnp.int32),
            pltpu.VMEM((2 * NROW, 128), jnp.float32),
            pltpu.VMEM((8192,), jnp.float32),
            pltpu.SemaphoreType.DMA,
            pltpu.SemaphoreType.DMA,
        ],
    )(_splat_body)


def kernel(point_clouds):
    pct = jnp.swapaxes(point_clouds, 1, 2)
    xs = pct[:, 0, :].reshape(B, 2048, 128)
    ys = pct[:, 1, :].reshape(B, 2048, 128)
    zs = pct[:, 2, :].reshape(B, 2048, 128)
    params = _params_tc(xs, ys, zs)
    params_sc = params[:, :, :16].reshape(-1)
    return _splat_sc()(xs, ys, zs, params_sc)
